# baseline (speedup 1.0000x reference)
import functools

import jax
import jax.numpy as jnp
from jax import lax
from jax.experimental import pallas as pl
from jax.experimental.pallas import tpu as pltpu
from jax.experimental.pallas import tpu_sc as plsc

ACC_SCALE = 0.02
MAX_VEL = 0.1

NC = 2
NS = 16


def _sc_degrees(src, dst, n):
    e = dst.shape[0]
    epw = e // (NC * NS)
    ch = 20000
    assert epw % ch == 0 and epw * NC * NS == e
    mesh = plsc.VectorSubcoreMesh(core_axis_name="c", subcore_axis_name="s")

    @functools.partial(
        pl.kernel,
        out_type=(
            jax.ShapeDtypeStruct((NC, n), jnp.float32),
            jax.ShapeDtypeStruct((NC, n), jnp.float32),
        ),
        mesh=mesh,
        compiler_params=pltpu.CompilerParams(use_tc_tiling_on_sc=False),
        scratch_types=[
            pltpu.VMEM((ch,), jnp.int32),
            pltpu.VMEM((ch,), jnp.int32),
            pltpu.VMEM((ch,), jnp.float32),
            pltpu.SemaphoreType.DMA,
            pltpu.SemaphoreType.DMA,
            pltpu.SemaphoreType.DMA,
            pltpu.SemaphoreType.DMA,
            pltpu.VMEM_SHARED((n,), jnp.float32),
            pltpu.VMEM_SHARED((n,), jnp.float32),
        ],
    )
    def deg_kernel(src_h, dst_h, zeros_h, ones_h, out_src, out_dst,
                   idx_s, idx_d, ones_v, sem0, sem1, sem2, sem3,
                   dsrc_sh, ddst_sh):
        c = lax.axis_index("c")
        s = lax.axis_index("s")

        @pl.when(s == 0)
        def _():
            pltpu.sync_copy(zeros_h, dsrc_sh)
            pltpu.sync_copy(zeros_h, ddst_sh)

        pltpu.sync_copy(ones_h, ones_v)
        plsc.subcore_barrier()

        base = (c * NS + s) * epw

        def body(i, carry):
            off = base + i * ch
            ld_s = pltpu.async_copy(src_h.at[pl.ds(off, ch)], idx_s, sem0)
            ld_d = pltpu.async_copy(dst_h.at[pl.ds(off, ch)], idx_d, sem1)
            ld_s.wait()
            sc_s = pltpu.async_copy(ones_v, dsrc_sh.at[idx_s], sem2, add=True)
            ld_d.wait()
            sc_d = pltpu.async_copy(ones_v, ddst_sh.at[idx_d], sem3, add=True)
            sc_s.wait()
            sc_d.wait()
            return carry

        lax.fori_loop(0, epw // ch, body, 0)
        plsc.subcore_barrier()

        @pl.when(s == 0)
        def _():
            pltpu.sync_copy(dsrc_sh, out_src.at[c])
            pltpu.sync_copy(ddst_sh, out_dst.at[c])

    zeros = jnp.zeros((n,), jnp.float32)
    ones = jnp.ones((ch,), jnp.float32)
    return deg_kernel(src, dst, zeros, ones)


def _sc_messages(src, dst, yt, n):
    e = src.shape[0]
    epw = e // (NC * NS)
    ch = 10000
    assert epw % (2 * ch) == 0
    mesh = plsc.VectorSubcoreMesh(core_axis_name="c", subcore_axis_name="s")

    @functools.partial(
        pl.kernel,
        out_type=jax.ShapeDtypeStruct((NC, 2, n), jnp.float32),
        mesh=mesh,
        compiler_params=pltpu.CompilerParams(use_tc_tiling_on_sc=False),
        scratch_types=[
            [pltpu.VMEM((ch,), jnp.int32)] * 4,
            [pltpu.VMEM((ch,), jnp.float32)] * 4,
            [pltpu.SemaphoreType.DMA] * 12,
            pltpu.VMEM_SHARED((n,), jnp.float32),
            pltpu.VMEM_SHARED((n,), jnp.float32),
            pltpu.VMEM_SHARED((n,), jnp.float32),
            pltpu.VMEM_SHARED((n,), jnp.float32),
        ],
    )
    def msg_kernel(src_h, dst_h, y0_h, y1_h, zeros_h, out_acc,
                   idxs, vs, sems, y0_sh, y1_sh, a0_sh, a1_sh):
        c = lax.axis_index("c")
        s = lax.axis_index("s")
        idx_s0, idx_s1, idx_d0, idx_d1 = idxs
        v00, v01, v10, v11 = vs

        @pl.when(s == 0)
        def _():
            pltpu.sync_copy(zeros_h, a0_sh)
            pltpu.sync_copy(zeros_h, a1_sh)

        plsc.subcore_barrier()

        base = (c * NS + s) * epw

        def body(j, carry):
            off0 = base + (2 * j) * ch
            off1 = off0 + ch
            l0s = pltpu.async_copy(src_h.at[pl.ds(off0, ch)], idx_s0, sems[0])
            l0d = pltpu.async_copy(dst_h.at[pl.ds(off0, ch)], idx_d0, sems[1])
            l1s = pltpu.async_copy(src_h.at[pl.ds(off1, ch)], idx_s1, sems[2])
            l1d = pltpu.async_copy(dst_h.at[pl.ds(off1, ch)], idx_d1, sems[3])
            l0s.wait()
            g00 = pltpu.async_copy(y0_h.at[idx_s0], v00, sems[4])
            g01 = pltpu.async_copy(y1_h.at[idx_s0], v01, sems[5])
            l1s.wait()
            g10 = pltpu.async_copy(y0_h.at[idx_s1], v10, sems[6])
            g11 = pltpu.async_copy(y1_h.at[idx_s1], v11, sems[7])
            g00.wait()
            g01.wait()
            l0d.wait()
            s00 = pltpu.async_copy(v00, a0_sh.at[idx_d0], sems[8], add=True)
            s01 = pltpu.async_copy(v01, a1_sh.at[idx_d0], sems[9], add=True)
            g10.wait()
            g11.wait()
            l1d.wait()
            s10 = pltpu.async_copy(v10, a0_sh.at[idx_d1], sems[10], add=True)
            s11 = pltpu.async_copy(v11, a1_sh.at[idx_d1], sems[11], add=True)
            s00.wait()
            s01.wait()
            s10.wait()
            s11.wait()
            return carry

        lax.fori_loop(0, epw // (2 * ch), body, 0)
        plsc.subcore_barrier()

        @pl.when(s == 0)
        def _():
            pltpu.sync_copy(a0_sh, out_acc.at[c, 0])
            pltpu.sync_copy(a1_sh, out_acc.at[c, 1])

    zeros = jnp.zeros((n,), jnp.float32)
    return msg_kernel(src, dst, yt[0], yt[1], zeros)


def _tc_mid_body(xt_ref, w_ref, degp_ref, yt_ref, dinv_ref):
    deg = degp_ref[0:1, :] + degp_ref[1:2, :] + 1.0
    dinv = lax.rsqrt(deg)
    dinv_ref[...] = dinv
    for j in range(2):
        xw = xt_ref[0:1, :] * w_ref[0, j]
        for cc in range(1, 5):
            xw = xw + xt_ref[cc:cc + 1, :] * w_ref[cc, j]
        yt_ref[j:j + 1, :] = xw * dinv


def _tc_mid(xt, w, degp, n):
    return pl.pallas_call(
        _tc_mid_body,
        out_shape=(
            jax.ShapeDtypeStruct((2, n), jnp.float32),
            jax.ShapeDtypeStruct((1, n), jnp.float32),
        ),
        in_specs=[
            pl.BlockSpec(memory_space=pltpu.VMEM),
            pl.BlockSpec(memory_space=pltpu.SMEM),
            pl.BlockSpec(memory_space=pltpu.VMEM),
        ],
        out_specs=(
            pl.BlockSpec(memory_space=pltpu.VMEM),
            pl.BlockSpec(memory_space=pltpu.VMEM),
        ),
    )(xt, w, degp)


def _tc_final_body(xt_ref, yt_ref, dinv_ref, acct_ref, degsp_ref, b_ref,
                   newxt_ref, vb_ref, pp_ref, bc_ref, fr_ref):
    n = xt_ref.shape[1]
    dinv = dinv_ref[0:1, :]
    food_mask = (xt_ref[4:5, :] == 1.0).astype(jnp.float32)
    vb = []
    pp = []
    bc = 0.0
    for j in range(2):
        acc = acct_ref[0, j:j + 1, :] + acct_ref[1, j:j + 1, :]
        h = dinv * (yt_ref[j:j + 1, :] + acc) + b_ref[j]
        a = h * ACC_SCALE * food_mask
        vel = jnp.clip(xt_ref[2 + j:3 + j, :] + a, -MAX_VEL, MAX_VEL)
        pos = xt_ref[j:j + 1, :] + vel
        newxt_ref[j:j + 1, :] = pos
        newxt_ref[2 + j:3 + j, :] = vel
        apos = jnp.abs(pos)
        bc = bc + jnp.sum(jnp.where(apos > 1.0, jnp.log(apos), 0.0))
        vb.append(jnp.sum(jnp.abs(vel)) / n)
        pp.append(jnp.sum(apos) / n)
    newxt_ref[4:5, :] = xt_ref[4:5, :]
    deg_src = degsp_ref[0:1, :] + degsp_ref[1:2, :]
    fr = jnp.sum(jnp.where((xt_ref[4:5, :] == 0.0) & (deg_src > 4.0),
                           1.0, 0.0))
    vb_ref[0] = vb[0]
    vb_ref[1] = vb[1]
    pp_ref[0] = pp[0]
    pp_ref[1] = pp[1]
    bc_ref[0] = bc
    fr_ref[0] = fr


def _tc_final(xt, yt, dinv, acct, degsp, b, n):
    return pl.pallas_call(
        _tc_final_body,
        out_shape=(
            jax.ShapeDtypeStruct((5, n), jnp.float32),
            jax.ShapeDtypeStruct((2,), jnp.float32),
            jax.ShapeDtypeStruct((2,), jnp.float32),
            jax.ShapeDtypeStruct((1,), jnp.float32),
            jax.ShapeDtypeStruct((1,), jnp.float32),
        ),
        in_specs=[
            pl.BlockSpec(memory_space=pltpu.VMEM),
            pl.BlockSpec(memory_space=pltpu.VMEM),
            pl.BlockSpec(memory_space=pltpu.VMEM),
            pl.BlockSpec(memory_space=pltpu.VMEM),
            pl.BlockSpec(memory_space=pltpu.VMEM),
            pl.BlockSpec(memory_space=pltpu.SMEM),
        ],
        out_specs=(
            pl.BlockSpec(memory_space=pltpu.VMEM),
            pl.BlockSpec(memory_space=pltpu.SMEM),
            pl.BlockSpec(memory_space=pltpu.SMEM),
            pl.BlockSpec(memory_space=pltpu.SMEM),
            pl.BlockSpec(memory_space=pltpu.SMEM),
        ),
    )(xt, yt, dinv, acct, degsp, b)


def kernel(x, edge_index, W, b):
    n = x.shape[0]
    src = edge_index[0]
    dst = edge_index[1]

    degsp, degdp = _sc_degrees(src, dst, n)

    xt = x.T
    yt, dinv = _tc_mid(xt, W, degdp, n)

    acct = _sc_messages(src, dst, yt, n)

    newxt, vb, pp, bc, fr = _tc_final(xt, yt, dinv, acct, degsp, b, n)
    return (newxt.T, vb, pp, bc[0], fr[0])


# speedup vs baseline: 1.8127x; 1.8127x over previous
import functools

import jax
import jax.numpy as jnp
from jax import lax
from jax.experimental import pallas as pl
from jax.experimental.pallas import tpu as pltpu
from jax.experimental.pallas import tpu_sc as plsc

ACC_SCALE = 0.02
MAX_VEL = 0.1

NC = 2
NS = 16


def _sc_degrees(src, dst, n):
    e = dst.shape[0]
    epw = e // (NC * NS)
    ch = 20000
    assert epw % ch == 0 and epw * NC * NS == e
    mesh = plsc.VectorSubcoreMesh(core_axis_name="c", subcore_axis_name="s")

    @functools.partial(
        pl.kernel,
        out_type=(
            jax.ShapeDtypeStruct((NC, n), jnp.float32),
            jax.ShapeDtypeStruct((NC, n), jnp.float32),
        ),
        mesh=mesh,
        compiler_params=pltpu.CompilerParams(use_tc_tiling_on_sc=False),
        scratch_types=[
            pltpu.VMEM((ch,), jnp.int32),
            pltpu.VMEM((ch,), jnp.int32),
            pltpu.VMEM((ch,), jnp.float32),
            pltpu.SemaphoreType.DMA,
            pltpu.SemaphoreType.DMA,
            pltpu.SemaphoreType.DMA,
            pltpu.SemaphoreType.DMA,
            pltpu.VMEM_SHARED((n,), jnp.float32),
            pltpu.VMEM_SHARED((n,), jnp.float32),
        ],
    )
    def deg_kernel(src_h, dst_h, zeros_h, ones_h, out_src, out_dst,
                   idx_s, idx_d, ones_v, sem0, sem1, sem2, sem3,
                   dsrc_sh, ddst_sh):
        c = lax.axis_index("c")
        s = lax.axis_index("s")

        @pl.when(s == 0)
        def _():
            pltpu.sync_copy(zeros_h, dsrc_sh)
            pltpu.sync_copy(zeros_h, ddst_sh)

        pltpu.sync_copy(ones_h, ones_v)
        plsc.subcore_barrier()

        base = (c * NS + s) * epw

        def body(i, carry):
            off = base + i * ch
            ld_s = pltpu.async_copy(src_h.at[pl.ds(off, ch)], idx_s, sem0)
            ld_d = pltpu.async_copy(dst_h.at[pl.ds(off, ch)], idx_d, sem1)
            ld_s.wait()
            sc_s = pltpu.async_copy(ones_v, dsrc_sh.at[idx_s], sem2, add=True)
            ld_d.wait()
            sc_d = pltpu.async_copy(ones_v, ddst_sh.at[idx_d], sem3, add=True)
            sc_s.wait()
            sc_d.wait()
            return carry

        lax.fori_loop(0, epw // ch, body, 0)
        plsc.subcore_barrier()

        @pl.when(s == 0)
        def _():
            pltpu.sync_copy(dsrc_sh, out_src.at[c])
            pltpu.sync_copy(ddst_sh, out_dst.at[c])

    zeros = jnp.zeros((n,), jnp.float32)
    ones = jnp.ones((ch,), jnp.float32)
    return deg_kernel(src, dst, zeros, ones)


def _sc_messages(src, dst, ypk, n):
    e = src.shape[0]
    epw = e // (NC * NS)
    ch = 10000
    assert epw % (2 * ch) == 0
    mesh = plsc.VectorSubcoreMesh(core_axis_name="c", subcore_axis_name="s")

    @functools.partial(
        pl.kernel,
        out_type=jax.ShapeDtypeStruct((NC, 2, n), jnp.float32),
        mesh=mesh,
        compiler_params=pltpu.CompilerParams(use_tc_tiling_on_sc=False, needs_layout_passes=False),
        scratch_types=[
            [pltpu.VMEM((ch,), jnp.int32)] * 4,
            [pltpu.VMEM((ch,), jnp.int32)] * 2,
            [pltpu.VMEM((ch,), jnp.float32)] * 4,
            [pltpu.SemaphoreType.DMA] * 10,
            pltpu.VMEM_SHARED((n,), jnp.int32),
            pltpu.VMEM_SHARED((n,), jnp.float32),
            pltpu.VMEM_SHARED((n,), jnp.float32),
        ],
    )
    def msg_kernel(src_h, dst_h, ypk_h, zeros_h, out_acc,
                   idxs, pks, vs, sems, ypk_sh, a0_sh, a1_sh):
        c = lax.axis_index("c")
        s = lax.axis_index("s")
        idx_s0, idx_s1, idx_d0, idx_d1 = idxs
        pk0, pk1 = pks
        v00, v01, v10, v11 = vs

        @pl.when(s == 0)
        def _():
            pltpu.sync_copy(ypk_h, ypk_sh)
            pltpu.sync_copy(zeros_h, a0_sh)
            pltpu.sync_copy(zeros_h, a1_sh)

        plsc.subcore_barrier()

        base = (c * NS + s) * epw
        mask_hi = jnp.int32(-65536)

        def unpack(pk, va, vb):
            def upk(k, carry):
                w = pk[pl.ds(k * 16, 16)]
                va[pl.ds(k * 16, 16)] = plsc.bitcast(
                    lax.bitwise_and(w, mask_hi), jnp.float32)
                vb[pl.ds(k * 16, 16)] = plsc.bitcast(
                    lax.shift_left(w, 16), jnp.float32)
                return carry
            lax.fori_loop(0, ch // 16, upk, 0)

        def body(j, carry):
            off0 = base + (2 * j) * ch
            off1 = off0 + ch
            l0s = pltpu.async_copy(src_h.at[pl.ds(off0, ch)], idx_s0, sems[0])
            l0d = pltpu.async_copy(dst_h.at[pl.ds(off0, ch)], idx_d0, sems[1])
            l1s = pltpu.async_copy(src_h.at[pl.ds(off1, ch)], idx_s1, sems[2])
            l1d = pltpu.async_copy(dst_h.at[pl.ds(off1, ch)], idx_d1, sems[3])
            l0s.wait()
            g0 = pltpu.async_copy(ypk_sh.at[idx_s0], pk0, sems[4])
            l1s.wait()
            g1 = pltpu.async_copy(ypk_sh.at[idx_s1], pk1, sems[5])
            g0.wait()
            unpack(pk0, v00, v01)
            l0d.wait()
            s00 = pltpu.async_copy(v00, a0_sh.at[idx_d0], sems[6], add=True)
            s01 = pltpu.async_copy(v01, a1_sh.at[idx_d0], sems[7], add=True)
            g1.wait()
            unpack(pk1, v10, v11)
            l1d.wait()
            s10 = pltpu.async_copy(v10, a0_sh.at[idx_d1], sems[8], add=True)
            s11 = pltpu.async_copy(v11, a1_sh.at[idx_d1], sems[9], add=True)
            s00.wait()
            s01.wait()
            s10.wait()
            s11.wait()
            return carry

        lax.fori_loop(0, epw // (2 * ch), body, 0)
        plsc.subcore_barrier()

        @pl.when(s == 0)
        def _():
            pltpu.sync_copy(a0_sh, out_acc.at[c, 0])
            pltpu.sync_copy(a1_sh, out_acc.at[c, 1])

    zeros = jnp.zeros((n,), jnp.float32)
    return msg_kernel(src, dst, ypk, zeros)


def _tc_mid_body(xt_ref, w_ref, degp_ref, yt_ref, dinv_ref, ypk_ref):
    deg = degp_ref[0:1, :] + degp_ref[1:2, :] + 1.0
    dinv = lax.rsqrt(deg)
    dinv_ref[...] = dinv
    ys = []
    for j in range(2):
        xw = xt_ref[0:1, :] * w_ref[0, j]
        for cc in range(1, 5):
            xw = xw + xt_ref[cc:cc + 1, :] * w_ref[cc, j]
        y = xw * dinv
        yt_ref[j:j + 1, :] = y
        ys.append(y)
    q0 = lax.bitcast_convert_type(ys[0], jnp.int32)
    q1 = lax.bitcast_convert_type(ys[1], jnp.int32)
    r0 = lax.bitwise_and(q0 + 0x8000, jnp.int32(-65536))
    r1 = lax.shift_right_logical(
        lax.bitwise_and(q1 + 0x8000, jnp.int32(-65536)), 16)
    ypk_ref[...] = lax.bitwise_or(r0, r1)


def _tc_mid(xt, w, degp, n):
    return pl.pallas_call(
        _tc_mid_body,
        out_shape=(
            jax.ShapeDtypeStruct((2, n), jnp.float32),
            jax.ShapeDtypeStruct((1, n), jnp.float32),
            jax.ShapeDtypeStruct((1, n), jnp.int32),
        ),
        in_specs=[
            pl.BlockSpec(memory_space=pltpu.VMEM),
            pl.BlockSpec(memory_space=pltpu.SMEM),
            pl.BlockSpec(memory_space=pltpu.VMEM),
        ],
        out_specs=(
            pl.BlockSpec(memory_space=pltpu.VMEM),
            pl.BlockSpec(memory_space=pltpu.VMEM),
            pl.BlockSpec(memory_space=pltpu.VMEM),
        ),
    )(xt, w, degp)


def _tc_final_body(xt_ref, yt_ref, dinv_ref, acct_ref, degsp_ref, b_ref,
                   newxt_ref, vb_ref, pp_ref, bc_ref, fr_ref):
    n = xt_ref.shape[1]
    dinv = dinv_ref[0:1, :]
    food_mask = (xt_ref[4:5, :] == 1.0).astype(jnp.float32)
    vb = []
    pp = []
    bc = 0.0
    for j in range(2):
        acc = acct_ref[0, j:j + 1, :] + acct_ref[1, j:j + 1, :]
        h = dinv * (yt_ref[j:j + 1, :] + acc) + b_ref[j]
        a = h * ACC_SCALE * food_mask
        vel = jnp.clip(xt_ref[2 + j:3 + j, :] + a, -MAX_VEL, MAX_VEL)
        pos = xt_ref[j:j + 1, :] + vel
        newxt_ref[j:j + 1, :] = pos
        newxt_ref[2 + j:3 + j, :] = vel
        apos = jnp.abs(pos)
        bc = bc + jnp.sum(jnp.where(apos > 1.0, jnp.log(apos), 0.0))
        vb.append(jnp.sum(jnp.abs(vel)) / n)
        pp.append(jnp.sum(apos) / n)
    newxt_ref[4:5, :] = xt_ref[4:5, :]
    deg_src = degsp_ref[0:1, :] + degsp_ref[1:2, :]
    fr = jnp.sum(jnp.where((xt_ref[4:5, :] == 0.0) & (deg_src > 4.0),
                           1.0, 0.0))
    vb_ref[0] = vb[0]
    vb_ref[1] = vb[1]
    pp_ref[0] = pp[0]
    pp_ref[1] = pp[1]
    bc_ref[0] = bc
    fr_ref[0] = fr


def _tc_final(xt, yt, dinv, acct, degsp, b, n):
    return pl.pallas_call(
        _tc_final_body,
        out_shape=(
            jax.ShapeDtypeStruct((5, n), jnp.float32),
            jax.ShapeDtypeStruct((2,), jnp.float32),
            jax.ShapeDtypeStruct((2,), jnp.float32),
            jax.ShapeDtypeStruct((1,), jnp.float32),
            jax.ShapeDtypeStruct((1,), jnp.float32),
        ),
        in_specs=[
            pl.BlockSpec(memory_space=pltpu.VMEM),
            pl.BlockSpec(memory_space=pltpu.VMEM),
            pl.BlockSpec(memory_space=pltpu.VMEM),
            pl.BlockSpec(memory_space=pltpu.VMEM),
            pl.BlockSpec(memory_space=pltpu.VMEM),
            pl.BlockSpec(memory_space=pltpu.SMEM),
        ],
        out_specs=(
            pl.BlockSpec(memory_space=pltpu.VMEM),
            pl.BlockSpec(memory_space=pltpu.SMEM),
            pl.BlockSpec(memory_space=pltpu.SMEM),
            pl.BlockSpec(memory_space=pltpu.SMEM),
            pl.BlockSpec(memory_space=pltpu.SMEM),
        ),
    )(xt, yt, dinv, acct, degsp, b)


def kernel(x, edge_index, W, b):
    n = x.shape[0]
    src = edge_index[0]
    dst = edge_index[1]

    degsp, degdp = _sc_degrees(src, dst, n)

    xt = x.T
    yt, dinv, ypk = _tc_mid(xt, W, degdp, n)

    acct = _sc_messages(src, dst, ypk[0], n)

    newxt, vb, pp, bc, fr = _tc_final(xt, yt, dinv, acct, degsp, b, n)
    return (newxt.T, vb, pp, bc[0], fr[0])


# speedup vs baseline: 1.8138x; 1.0006x over previous
import functools

import jax
import jax.numpy as jnp
from jax import lax
from jax.experimental import pallas as pl
from jax.experimental.pallas import tpu as pltpu
from jax.experimental.pallas import tpu_sc as plsc

ACC_SCALE = 0.02
MAX_VEL = 0.1

NC = 2
NS = 16


def _sc_degrees(src, dst, n):
    e = dst.shape[0]
    epw = e // (NC * NS)
    ch = 25000
    assert epw % ch == 0 and epw * NC * NS == e
    mesh = plsc.VectorSubcoreMesh(core_axis_name="c", subcore_axis_name="s")

    @functools.partial(
        pl.kernel,
        out_type=(
            jax.ShapeDtypeStruct((NC, n), jnp.float32),
            jax.ShapeDtypeStruct((NC, n), jnp.float32),
        ),
        mesh=mesh,
        compiler_params=pltpu.CompilerParams(use_tc_tiling_on_sc=False),
        scratch_types=[
            pltpu.VMEM((ch,), jnp.int32),
            pltpu.VMEM((ch,), jnp.int32),
            pltpu.VMEM((ch,), jnp.float32),
            pltpu.SemaphoreType.DMA,
            pltpu.SemaphoreType.DMA,
            pltpu.SemaphoreType.DMA,
            pltpu.SemaphoreType.DMA,
            pltpu.VMEM_SHARED((n,), jnp.float32),
            pltpu.VMEM_SHARED((n,), jnp.float32),
        ],
    )
    def deg_kernel(src_h, dst_h, zeros_h, ones_h, out_src, out_dst,
                   idx_s, idx_d, ones_v, sem0, sem1, sem2, sem3,
                   dsrc_sh, ddst_sh):
        c = lax.axis_index("c")
        s = lax.axis_index("s")

        @pl.when(s == 0)
        def _():
            pltpu.sync_copy(zeros_h, dsrc_sh)
            pltpu.sync_copy(zeros_h, ddst_sh)

        pltpu.sync_copy(ones_h, ones_v)
        plsc.subcore_barrier()

        base = (c * NS + s) * epw

        def body(i, carry):
            off = base + i * ch
            ld_s = pltpu.async_copy(src_h.at[pl.ds(off, ch)], idx_s, sem0)
            ld_d = pltpu.async_copy(dst_h.at[pl.ds(off, ch)], idx_d, sem1)
            ld_s.wait()
            sc_s = pltpu.async_copy(ones_v, dsrc_sh.at[idx_s], sem2, add=True)
            ld_d.wait()
            sc_d = pltpu.async_copy(ones_v, ddst_sh.at[idx_d], sem3, add=True)
            sc_s.wait()
            sc_d.wait()
            return carry

        lax.fori_loop(0, epw // ch, body, 0)
        plsc.subcore_barrier()

        @pl.when(s == 0)
        def _():
            pltpu.sync_copy(dsrc_sh, out_src.at[c])
            pltpu.sync_copy(ddst_sh, out_dst.at[c])

    zeros = jnp.zeros((n,), jnp.float32)
    ones = jnp.ones((ch,), jnp.float32)
    return deg_kernel(src, dst, zeros, ones)


def _sc_messages(src, dst, ypk, n):
    e = src.shape[0]
    epw = e // (NC * NS)
    ch = 10000
    assert epw % (2 * ch) == 0
    mesh = plsc.VectorSubcoreMesh(core_axis_name="c", subcore_axis_name="s")

    @functools.partial(
        pl.kernel,
        out_type=jax.ShapeDtypeStruct((NC, 2, n), jnp.float32),
        mesh=mesh,
        compiler_params=pltpu.CompilerParams(use_tc_tiling_on_sc=False, needs_layout_passes=False),
        scratch_types=[
            [pltpu.VMEM((ch,), jnp.int32)] * 4,
            [pltpu.VMEM((ch,), jnp.int32)] * 2,
            [pltpu.VMEM((ch,), jnp.float32)] * 4,
            [pltpu.SemaphoreType.DMA] * 10,
            pltpu.VMEM_SHARED((n,), jnp.int32),
            pltpu.VMEM_SHARED((n,), jnp.float32),
            pltpu.VMEM_SHARED((n,), jnp.float32),
        ],
    )
    def msg_kernel(src_h, dst_h, ypk_h, zeros_h, out_acc,
                   idxs, pks, vs, sems, ypk_sh, a0_sh, a1_sh):
        c = lax.axis_index("c")
        s = lax.axis_index("s")
        idx_s0, idx_s1, idx_d0, idx_d1 = idxs
        pk0, pk1 = pks
        v00, v01, v10, v11 = vs

        @pl.when(s == 0)
        def _():
            pltpu.sync_copy(ypk_h, ypk_sh)
            pltpu.sync_copy(zeros_h, a0_sh)
            pltpu.sync_copy(zeros_h, a1_sh)

        plsc.subcore_barrier()

        base = (c * NS + s) * epw
        mask_hi = jnp.int32(-65536)

        def unpack(pk, va, vb):
            def upk(k, carry):
                w = pk[pl.ds(k * 16, 16)]
                va[pl.ds(k * 16, 16)] = plsc.bitcast(
                    lax.bitwise_and(w, mask_hi), jnp.float32)
                vb[pl.ds(k * 16, 16)] = plsc.bitcast(
                    lax.shift_left(w, 16), jnp.float32)
                return carry
            lax.fori_loop(0, ch // 16, upk, 0)

        def body(j, carry):
            off0 = base + (2 * j) * ch
            off1 = off0 + ch
            l0s = pltpu.async_copy(src_h.at[pl.ds(off0, ch)], idx_s0, sems[0])
            l0d = pltpu.async_copy(dst_h.at[pl.ds(off0, ch)], idx_d0, sems[1])
            l1s = pltpu.async_copy(src_h.at[pl.ds(off1, ch)], idx_s1, sems[2])
            l1d = pltpu.async_copy(dst_h.at[pl.ds(off1, ch)], idx_d1, sems[3])
            l0s.wait()
            g0 = pltpu.async_copy(ypk_sh.at[idx_s0], pk0, sems[4])
            l1s.wait()
            g1 = pltpu.async_copy(ypk_sh.at[idx_s1], pk1, sems[5])
            g0.wait()
            unpack(pk0, v00, v01)
            l0d.wait()
            s00 = pltpu.async_copy(v00, a0_sh.at[idx_d0], sems[6], add=True)
            s01 = pltpu.async_copy(v01, a1_sh.at[idx_d0], sems[7], add=True)
            g1.wait()
            unpack(pk1, v10, v11)
            l1d.wait()
            s10 = pltpu.async_copy(v10, a0_sh.at[idx_d1], sems[8], add=True)
            s11 = pltpu.async_copy(v11, a1_sh.at[idx_d1], sems[9], add=True)
            s00.wait()
            s01.wait()
            s10.wait()
            s11.wait()
            return carry

        lax.fori_loop(0, epw // (2 * ch), body, 0)
        plsc.subcore_barrier()

        @pl.when(s == 0)
        def _():
            pltpu.sync_copy(a0_sh, out_acc.at[c, 0])
            pltpu.sync_copy(a1_sh, out_acc.at[c, 1])

    zeros = jnp.zeros((n,), jnp.float32)
    return msg_kernel(src, dst, ypk, zeros)


def _tc_mid_body(xt_ref, w_ref, degp_ref, yt_ref, dinv_ref, ypk_ref):
    deg = degp_ref[0:1, :] + degp_ref[1:2, :] + 1.0
    dinv = lax.rsqrt(deg)
    dinv_ref[...] = dinv
    ys = []
    for j in range(2):
        xw = xt_ref[0:1, :] * w_ref[0, j]
        for cc in range(1, 5):
            xw = xw + xt_ref[cc:cc + 1, :] * w_ref[cc, j]
        y = xw * dinv
        yt_ref[j:j + 1, :] = y
        ys.append(y)
    q0 = lax.bitcast_convert_type(ys[0], jnp.int32)
    q1 = lax.bitcast_convert_type(ys[1], jnp.int32)
    r0 = lax.bitwise_and(q0 + 0x8000, jnp.int32(-65536))
    r1 = lax.shift_right_logical(
        lax.bitwise_and(q1 + 0x8000, jnp.int32(-65536)), 16)
    ypk_ref[...] = lax.bitwise_or(r0, r1)


def _tc_mid(xt, w, degp, n):
    return pl.pallas_call(
        _tc_mid_body,
        out_shape=(
            jax.ShapeDtypeStruct((2, n), jnp.float32),
            jax.ShapeDtypeStruct((1, n), jnp.float32),
            jax.ShapeDtypeStruct((1, n), jnp.int32),
        ),
        in_specs=[
            pl.BlockSpec(memory_space=pltpu.VMEM),
            pl.BlockSpec(memory_space=pltpu.SMEM),
            pl.BlockSpec(memory_space=pltpu.VMEM),
        ],
        out_specs=(
            pl.BlockSpec(memory_space=pltpu.VMEM),
            pl.BlockSpec(memory_space=pltpu.VMEM),
            pl.BlockSpec(memory_space=pltpu.VMEM),
        ),
    )(xt, w, degp)


def _tc_final_body(xt_ref, yt_ref, dinv_ref, acct_ref, degsp_ref, b_ref,
                   newxt_ref, vb_ref, pp_ref, bc_ref, fr_ref):
    n = xt_ref.shape[1]
    dinv = dinv_ref[0:1, :]
    food_mask = (xt_ref[4:5, :] == 1.0).astype(jnp.float32)
    vb = []
    pp = []
    bc = 0.0
    for j in range(2):
        acc = acct_ref[0, j:j + 1, :] + acct_ref[1, j:j + 1, :]
        h = dinv * (yt_ref[j:j + 1, :] + acc) + b_ref[j]
        a = h * ACC_SCALE * food_mask
        vel = jnp.clip(xt_ref[2 + j:3 + j, :] + a, -MAX_VEL, MAX_VEL)
        pos = xt_ref[j:j + 1, :] + vel
        newxt_ref[j:j + 1, :] = pos
        newxt_ref[2 + j:3 + j, :] = vel
        apos = jnp.abs(pos)
        bc = bc + jnp.sum(jnp.where(apos > 1.0, jnp.log(apos), 0.0))
        vb.append(jnp.sum(jnp.abs(vel)) / n)
        pp.append(jnp.sum(apos) / n)
    newxt_ref[4:5, :] = xt_ref[4:5, :]
    deg_src = degsp_ref[0:1, :] + degsp_ref[1:2, :]
    fr = jnp.sum(jnp.where((xt_ref[4:5, :] == 0.0) & (deg_src > 4.0),
                           1.0, 0.0))
    vb_ref[0] = vb[0]
    vb_ref[1] = vb[1]
    pp_ref[0] = pp[0]
    pp_ref[1] = pp[1]
    bc_ref[0] = bc
    fr_ref[0] = fr


def _tc_final(xt, yt, dinv, acct, degsp, b, n):
    return pl.pallas_call(
        _tc_final_body,
        out_shape=(
            jax.ShapeDtypeStruct((5, n), jnp.float32),
            jax.ShapeDtypeStruct((2,), jnp.float32),
            jax.ShapeDtypeStruct((2,), jnp.float32),
            jax.ShapeDtypeStruct((1,), jnp.float32),
            jax.ShapeDtypeStruct((1,), jnp.float32),
        ),
        in_specs=[
            pl.BlockSpec(memory_space=pltpu.VMEM),
            pl.BlockSpec(memory_space=pltpu.VMEM),
            pl.BlockSpec(memory_space=pltpu.VMEM),
            pl.BlockSpec(memory_space=pltpu.VMEM),
            pl.BlockSpec(memory_space=pltpu.VMEM),
            pl.BlockSpec(memory_space=pltpu.SMEM),
        ],
        out_specs=(
            pl.BlockSpec(memory_space=pltpu.VMEM),
            pl.BlockSpec(memory_space=pltpu.SMEM),
            pl.BlockSpec(memory_space=pltpu.SMEM),
            pl.BlockSpec(memory_space=pltpu.SMEM),
            pl.BlockSpec(memory_space=pltpu.SMEM),
        ),
    )(xt, yt, dinv, acct, degsp, b)


def kernel(x, edge_index, W, b):
    n = x.shape[0]
    src = edge_index[0]
    dst = edge_index[1]

    degsp, degdp = _sc_degrees(src, dst, n)

    xt = x.T
    yt, dinv, ypk = _tc_mid(xt, W, degdp, n)

    acct = _sc_messages(src, dst, ypk[0], n)

    newxt, vb, pp, bc, fr = _tc_final(xt, yt, dinv, acct, degsp, b, n)
    return (newxt.T, vb, pp, bc[0], fr[0])
